# Initial kernel scaffold; baseline (speedup 1.0000x reference)
import jax
import jax.numpy as jnp
from jax.experimental import pallas as pl


def kernel(x, edge_index, batch, gin_w1, gin_b1, gin_w2, gin_b2, attn_win, attn_bin, attn_wout, attn_bout, mlp_w1, mlp_b1, mlp_w2, mlp_b2, norm_gamma, norm_beta):
    raise NotImplementedError("write your pallas kernel here")


# speedup vs baseline: 7.0354x; 7.0354x over previous
import functools

import jax
import jax.numpy as jnp
from jax import lax
from jax.experimental import pallas as pl
from jax.experimental.pallas import tpu as pltpu
from jax.experimental.pallas import tpu_sc as plsc

_N = 10000
_E = 320000
_D = 128
_L = 8
_H = 8
_HD = _D // _H
_NG = 50
_BQ = 200
_NB = _N // _BQ

_NC = 2
_NS = 16
_NW = _NC * _NS
_EPW = _E // _NW
_CH = 80
_NCH = _EPW // _CH


def _sc_scatter_add(x, src, dst, zeros_nd):
    mesh = plsc.VectorSubcoreMesh(core_axis_name="c", subcore_axis_name="s")

    @functools.partial(
        pl.kernel,
        out_type=jax.ShapeDtypeStruct((_NC, _N, _D), jnp.float32),
        mesh=mesh,
        scratch_types=[
            pltpu.VMEM((_CH,), jnp.int32),
            pltpu.VMEM((_CH,), jnp.int32),
            pltpu.VMEM((_CH, _D), jnp.float32),
            pltpu.VMEM_SHARED((_N, _D), jnp.float32),
            pltpu.SemaphoreType.DMA,
        ],
    )
    def body(x_hbm, src_hbm, dst_hbm, z_hbm, out_hbm, sidx, didx, rows, acc, sem):
        c = lax.axis_index("c")
        s = lax.axis_index("s")
        wid = c * _NS + s

        @pl.when(s == 0)
        def _():
            pltpu.sync_copy(z_hbm, acc)

        plsc.subcore_barrier()

        base = wid * _EPW

        def step(i, carry):
            off = pl.multiple_of(base + i * _CH, 8)
            pltpu.sync_copy(src_hbm.at[pl.ds(off, _CH)], sidx)
            pltpu.sync_copy(dst_hbm.at[pl.ds(off, _CH)], didx)
            pltpu.async_copy(x_hbm.at[sidx], rows, sem).wait()
            pltpu.sync_copy(rows, acc.at[didx], add=True)
            return carry

        lax.fori_loop(0, _NCH, step, 0)
        plsc.subcore_barrier()

        @pl.when(s == 0)
        def _():
            pltpu.sync_copy(acc, out_hbm.at[c])

    return body(x, src, dst, zeros_nd)


def _qkv_kernel(x_ref, w_ref, b_ref, o_ref):
    o_ref[...] = (
        jnp.dot(x_ref[...], w_ref[...], preferred_element_type=jnp.float32)
        + b_ref[0:1, :]
    )


def _qkv(x, winT, bin8):
    return pl.pallas_call(
        _qkv_kernel,
        grid=(10,),
        in_specs=[
            pl.BlockSpec((_N // 10, _D), lambda i: (i, 0)),
            pl.BlockSpec((_D, 3 * _D), lambda i: (0, 0)),
            pl.BlockSpec((8, 3 * _D), lambda i: (0, 0)),
        ],
        out_specs=pl.BlockSpec((_N // 10, 3 * _D), lambda i: (i, 0)),
        out_shape=jax.ShapeDtypeStruct((_N, 3 * _D), jnp.float32),
    )(x, winT, bin8)


def _attn_kernel(ranges_ref, qkv_ref, bq_ref, br_ref, o_ref):
    i = pl.program_id(0)
    kb_lo = ranges_ref[2 * i]
    kb_n = ranges_ref[2 * i + 1]
    scale = 1.0 / (_HD ** 0.5)

    q = qkv_ref[pl.ds(pl.multiple_of(i * _BQ, 8), _BQ), 0:_D] * scale
    bq = bq_ref[...]

    def step(j, carry):
        kb = kb_lo + j
        off = pl.multiple_of(kb * _BQ, 8)
        kk = qkv_ref[pl.ds(off, _BQ), _D:2 * _D]
        vv = qkv_ref[pl.ds(off, _BQ), 2 * _D:3 * _D]
        bk = br_ref[pl.ds(kb, 1), :, :].reshape(1, _BQ)
        msk = bq == bk
        new = []
        for h in range(_H):
            o, m, l = carry[h]
            sl = slice(h * _HD, (h + 1) * _HD)
            s = lax.dot_general(q[:, sl], kk[:, sl], (((1,), (1,)), ((), ())),
                                preferred_element_type=jnp.float32)
            s = jnp.where(msk, s, -1e9)
            m2 = jnp.maximum(m, jnp.max(s, axis=1, keepdims=True))
            p = jnp.exp(s - m2)
            alpha = jnp.exp(m - m2)
            l2 = l * alpha + jnp.sum(p, axis=1, keepdims=True)
            o2 = o * alpha + jnp.dot(p, vv[:, sl], preferred_element_type=jnp.float32)
            new.append((o2, m2, l2))
        return tuple(new)

    init = tuple(
        (jnp.zeros((_BQ, _HD), jnp.float32),
         jnp.full((_BQ, 1), -1e30, jnp.float32),
         jnp.zeros((_BQ, 1), jnp.float32))
        for _ in range(_H)
    )
    fin = lax.fori_loop(0, kb_n, step, init)
    o_ref[...] = jnp.concatenate([o / l for (o, m, l) in fin], axis=1)


def _attention(qkv, b_col, b_row, ranges):
    grid_spec = pltpu.PrefetchScalarGridSpec(
        num_scalar_prefetch=1,
        grid=(_NB,),
        in_specs=[
            pl.BlockSpec((_N, 3 * _D), lambda i, r: (0, 0)),
            pl.BlockSpec((_BQ, 1), lambda i, r: (i, 0)),
            pl.BlockSpec((_NB, 1, _BQ), lambda i, r: (0, 0, 0)),
        ],
        out_specs=pl.BlockSpec((_BQ, _D), lambda i, r: (i, 0)),
    )
    return pl.pallas_call(
        _attn_kernel,
        grid_spec=grid_spec,
        out_shape=jax.ShapeDtypeStruct((_N, _D), jnp.float32),
    )(ranges, qkv, b_col, b_row)


def _bn(h, g, b, eps=1e-5):
    mean = jnp.mean(h, axis=0, keepdims=True)
    var = jnp.mean((h - mean) ** 2, axis=0, keepdims=True)
    return g * (h - mean) / jnp.sqrt(var + eps) + b


def _dense_kernel(x_ref, a0_ref, a1_ref, at_ref,
                  gw1_ref, gb1_ref, gw2_ref, gb2_ref,
                  woT_ref, bo_ref, mw1_ref, mb1_ref, mw2_ref, mb2_ref,
                  nrm_ref, o_ref):
    x = x_ref[...]
    h = x + a0_ref[...] + a1_ref[...]
    t = jax.nn.relu(jnp.dot(h, gw1_ref[...], preferred_element_type=jnp.float32)
                    + gb1_ref[0:1, :])
    t = jnp.dot(t, gw2_ref[...], preferred_element_type=jnp.float32) + gb2_ref[0:1, :]
    t = t + x
    h1 = _bn(t, nrm_ref[0:1, :], nrm_ref[3:4, :])

    a = jnp.dot(at_ref[...], woT_ref[...], preferred_element_type=jnp.float32) + bo_ref[0:1, :]
    h2 = _bn(a + x, nrm_ref[1:2, :], nrm_ref[4:5, :])

    out = h1 + h2
    m = jax.nn.relu(jnp.dot(out, mw1_ref[...], preferred_element_type=jnp.float32)
                    + mb1_ref[0:1, :_D * 2])
    m = jnp.dot(m, mw2_ref[...], preferred_element_type=jnp.float32) + mb2_ref[0:1, :]
    out = out + m
    o_ref[...] = _bn(out, nrm_ref[2:3, :], nrm_ref[5:6, :])


def _dense(x, a0, a1, attn_raw, gw1, gb1, gw2, gb2, woT, bo, mw1, mb1, mw2, mb2, nrm):
    full = lambda shp: pl.BlockSpec(shp, lambda: tuple(0 for _ in shp))
    args = (x, a0, a1, attn_raw, gw1, gb1, gw2, gb2, woT, bo, mw1, mb1, mw2, mb2, nrm)
    return pl.pallas_call(
        _dense_kernel,
        grid=(),
        in_specs=[full(a.shape) for a in args],
        out_specs=full((_N, _D)),
        out_shape=jax.ShapeDtypeStruct((_N, _D), jnp.float32),
    )(*args)


def _pad8(b):
    return jnp.broadcast_to(b[None, :], (8, b.shape[0]))


def kernel(x, edge_index, batch, gin_w1, gin_b1, gin_w2, gin_b2,
           attn_win, attn_bin, attn_wout, attn_bout,
           mlp_w1, mlp_b1, mlp_w2, mlp_b2, norm_gamma, norm_beta):
    src = edge_index[0]
    dst = edge_index[1]
    zeros_nd = jnp.zeros((_N, _D), jnp.float32)

    starts = jnp.searchsorted(batch, jnp.arange(_NG + 1, dtype=jnp.int32))
    gf = batch[::_BQ]
    gl = batch[_BQ - 1::_BQ]
    k_lo = starts[gf]
    k_hi = starts[gl + 1]
    kb_lo = (k_lo // _BQ).astype(jnp.int32)
    kb_n = ((k_hi + _BQ - 1) // _BQ).astype(jnp.int32) - kb_lo
    ranges = jnp.stack([kb_lo, kb_n], axis=1).reshape(-1)

    b_col = batch[:, None]
    b_row = batch.reshape(_NB, 1, _BQ)

    winT = jnp.transpose(attn_win, (0, 2, 1))
    woutT = jnp.transpose(attn_wout, (0, 2, 1))
    nrm = jnp.concatenate([norm_gamma, norm_beta], axis=1)
    nrm = jnp.pad(nrm, ((0, 0), (0, 2), (0, 0)))

    for l in range(_L):
        agg = _sc_scatter_add(x, src, dst, zeros_nd)
        qkv = _qkv(x, winT[l], _pad8(attn_bin[l]))
        attn_raw = _attention(qkv, b_col, b_row, ranges)
        x = _dense(x, agg[0], agg[1], attn_raw,
                   gin_w1[l], _pad8(gin_b1[l]), gin_w2[l], _pad8(gin_b2[l]),
                   woutT[l], _pad8(attn_bout[l]),
                   mlp_w1[l], _pad8(mlp_b1[l]), mlp_w2[l], _pad8(mlp_b2[l]),
                   nrm[l])
    return x
